# baseline (speedup 1.0000x reference)
import functools

import jax
import jax.numpy as jnp
from jax import lax
from jax.experimental import pallas as pl
from jax.experimental.pallas import tpu as pltpu
from jax.experimental.pallas import tpu_sc as plsc

N = 10000
E = 320000
D = 128
NC = 2
NS = 16
NW = NC * NS
CB = 128
CPWK = 79
EPW = E // NW
TRASH = 10016
DB = 80
DCPW = E // NW // DB
NPAD = NS * 640
RPS = 640
ZR = 128

_MESH = plsc.VectorSubcoreMesh(core_axis_name="c", subcore_axis_name="s")


@functools.partial(
    pl.kernel,
    out_type=jax.ShapeDtypeStruct((NC * NPAD,), jnp.float32),
    mesh=_MESH,
    scratch_types=[
        pltpu.VMEM((DCPW, DB), jnp.int32),
        pltpu.VMEM((DB,), jnp.float32),
        pltpu.VMEM((DB,), jnp.float32),
        pltpu.VMEM_SHARED((NPAD,), jnp.float32),
    ],
)
def _deg_call(dst_hbm, out_hbm, dstv, onesv, zerosv, acc):
    c = lax.axis_index("c")
    s = lax.axis_index("s")
    w = s * NC + c
    for i in range(DB // 16):
        onesv[pl.ds(i * 16, 16)] = jnp.ones((16,), jnp.float32)
        zerosv[pl.ds(i * 16, 16)] = jnp.zeros((16,), jnp.float32)
    for k in range(640 // DB):
        pltpu.sync_copy(zerosv, acc.at[pl.ds(s * 640 + k * DB, DB)])
    plsc.subcore_barrier()
    pltpu.sync_copy(dst_hbm.at[w], dstv)

    def body(j, carry):
        pltpu.sync_copy(onesv, acc.at[dstv.at[j]], add=True)
        return carry

    lax.fori_loop(0, DCPW, body, 0)
    plsc.subcore_barrier()
    pltpu.sync_copy(
        acc.at[pl.ds(s * 640, 640)], out_hbm.at[pl.ds(c * NPAD + s * 640, 640)]
    )


@functools.partial(
    pl.kernel,
    out_type=jax.ShapeDtypeStruct((NC, NPAD, D), jnp.float32),
    mesh=_MESH,
    scratch_types=[
        pltpu.VMEM((3, 2, CB), jnp.int32),
        pltpu.VMEM((CB, D), jnp.float32),
        pltpu.VMEM((CB, D), jnp.float32),
        pltpu.VMEM_SHARED((NPAD, D), jnp.float32),
        pltpu.SemaphoreType.DMA,
        pltpu.SemaphoreType.DMA,
        pltpu.SemaphoreType.DMA,
        pltpu.SemaphoreType.DMA,
        pltpu.SemaphoreType.DMA,
    ],
)
def _agg_call(g_hbm, sd_hbm, out_hbm, idx, rows_a, rows_b, acc, i0, i1, i2, r0, r1):
    c = lax.axis_index("c")
    s = lax.axis_index("s")
    w = s * NC + c
    rows = (rows_a, rows_b)
    rsem = (r0, r1)
    isem = (i0, i1, i2)
    zv = jnp.zeros((16,), jnp.float32)

    def zbody(i, carry):
        for j in range(D // 16):
            rows_a[i, pl.ds(j * 16, 16)] = zv
        return carry

    lax.fori_loop(0, CB, zbody, 0)
    for k in range(RPS // CB):
        pltpu.sync_copy(rows_a, acc.at[pl.ds(s * RPS + k * CB, CB)])
    plsc.subcore_barrier()
    pltpu.sync_copy(sd_hbm.at[w, 0], idx.at[0])
    pltpu.async_copy(g_hbm.at[idx.at[0, 0]], rows_a, r0)
    pltpu.async_copy(sd_hbm.at[w, 1], idx.at[1], i1)

    def body(i, carry):
        for k in range(6):
            j = 6 * i + k
            sj, sj1, sj2 = k % 3, (k + 1) % 3, (k + 2) % 3
            pj, pj1 = k % 2, (k + 1) % 2
            pltpu.make_async_copy(g_hbm.at[idx.at[sj, 0]], rows[pj], rsem[pj]).wait()
            jn = jnp.minimum(j + 2, CPWK - 1)
            pltpu.async_copy(sd_hbm.at[w, jn], idx.at[sj2], isem[sj2])
            pltpu.make_async_copy(sd_hbm.at[w, j + 1], idx.at[sj1], isem[sj1]).wait()
            pltpu.async_copy(g_hbm.at[idx.at[sj1, 0]], rows[pj1], rsem[pj1])
            pltpu.sync_copy(rows[pj], acc.at[idx.at[sj, 1]], add=True)
        return carry

    lax.fori_loop(0, (CPWK - 1) // 6, body, 0)
    pltpu.make_async_copy(g_hbm.at[idx.at[0, 0]], rows_a, r0).wait()
    pltpu.sync_copy(rows_a, acc.at[idx.at[0, 1]], add=True)
    pltpu.make_async_copy(sd_hbm.at[w, CPWK - 1], idx.at[1], i1).wait()
    plsc.subcore_barrier()
    for k in range(RPS // CB):
        base = s * RPS + k * CB
        pltpu.sync_copy(acc.at[pl.ds(base, CB)], out_hbm.at[c, pl.ds(base, CB)])


BM = 1000


def _mm_body(x_ref, w_ref, o_ref):
    o_ref[...] = jnp.dot(x_ref[...], w_ref[...], preferred_element_type=jnp.float32)


def _mm(x, w):
    return pl.pallas_call(
        _mm_body,
        grid=(N // BM,),
        in_specs=[
            pl.BlockSpec((BM, D), lambda i: (i, 0)),
            pl.BlockSpec((D, D), lambda i: (0, 0)),
        ],
        out_specs=pl.BlockSpec((BM, D), lambda i: (i, 0)),
        out_shape=jax.ShapeDtypeStruct((N, D), jnp.float32),
    )(x, w)


def _prep_body(dega_ref, degb_ref, xw_ref, dinv_ref, g_ref):
    deg = dega_ref[...] + degb_ref[...]
    dinv = jnp.where(deg > 0, 1.0 / jnp.sqrt(jnp.maximum(deg, 1.0)), 0.0)
    dinv_ref[...] = dinv
    g_ref[...] = xw_ref[...] * dinv


def _prep(dega, degb, xw):
    return pl.pallas_call(
        _prep_body,
        grid=(N // BM,),
        in_specs=[
            pl.BlockSpec((BM, 1), lambda i: (i, 0)),
            pl.BlockSpec((BM, 1), lambda i: (i, 0)),
            pl.BlockSpec((BM, D), lambda i: (i, 0)),
        ],
        out_specs=[
            pl.BlockSpec((BM, 1), lambda i: (i, 0)),
            pl.BlockSpec((BM, D), lambda i: (i, 0)),
        ],
        out_shape=[
            jax.ShapeDtypeStruct((N, 1), jnp.float32),
            jax.ShapeDtypeStruct((N, D), jnp.float32),
        ],
    )(dega, degb, xw)


def _mid_body(s1_ref, dinv_ref, b1_ref, w2_ref, h1_ref, g2_ref):
    stot = s1_ref[0] + s1_ref[1]
    h1 = jnp.maximum(stot * dinv_ref[...] + b1_ref[...], 0.0)
    h1_ref[...] = h1
    g2_ref[...] = (
        jnp.dot(h1, w2_ref[...], preferred_element_type=jnp.float32) * dinv_ref[...]
    )


def _mid(s1, dinv, b1, w2):
    return pl.pallas_call(
        _mid_body,
        grid=(N // BM,),
        in_specs=[
            pl.BlockSpec((NC, BM, D), lambda i: (0, i, 0)),
            pl.BlockSpec((BM, 1), lambda i: (i, 0)),
            pl.BlockSpec((1, D), lambda i: (0, 0)),
            pl.BlockSpec((D, D), lambda i: (0, 0)),
        ],
        out_specs=[
            pl.BlockSpec((BM, D), lambda i: (i, 0)),
            pl.BlockSpec((BM, D), lambda i: (i, 0)),
        ],
        out_shape=[
            jax.ShapeDtypeStruct((N, D), jnp.float32),
            jax.ShapeDtypeStruct((N, D), jnp.float32),
        ],
    )(s1, dinv, b1, w2)


def _out_body(s2_ref, dinv_ref, b2_ref, h2_ref):
    h2_ref[...] = (s2_ref[0] + s2_ref[1]) * dinv_ref[...] + b2_ref[...]


def _out(s2, dinv, b2):
    return pl.pallas_call(
        _out_body,
        grid=(N // BM,),
        in_specs=[
            pl.BlockSpec((NC, BM, D), lambda i: (0, i, 0)),
            pl.BlockSpec((BM, 1), lambda i: (i, 0)),
            pl.BlockSpec((1, D), lambda i: (0, 0)),
        ],
        out_specs=pl.BlockSpec((BM, D), lambda i: (i, 0)),
        out_shape=jax.ShapeDtypeStruct((N, D), jnp.float32),
    )(s2, dinv, b2)


def kernel(x, edge_index, W1, b1, W2, b2):
    pad = CPWK * CB - EPW
    src_p = jnp.pad(
        edge_index[0].reshape(NW, EPW), ((0, 0), (0, pad)), constant_values=0
    ).reshape(NW, CPWK, CB)
    dst_p = jnp.pad(
        edge_index[1].reshape(NW, EPW), ((0, 0), (0, pad)), constant_values=TRASH
    ).reshape(NW, CPWK, CB)
    sd = jnp.stack([src_p, dst_p], axis=2)
    dst3d = edge_index[1].reshape(NW, DCPW, DB)
    deg1d = _deg_call(dst3d)
    dega = deg1d[:N].reshape(N, 1)
    degb = deg1d[NPAD : NPAD + N].reshape(N, 1)
    xw1 = _mm(x, W1)
    dinv, g1 = _prep(dega, degb, xw1)
    s1 = _agg_call(g1, sd)
    h1, g2 = _mid(s1, dinv, b1.reshape(1, D), W2)
    s2 = _agg_call(g2, sd)
    h2 = _out(s2, dinv, b2.reshape(1, D))
    return (h1, h2)


# speedup vs baseline: 1.8217x; 1.8217x over previous
import functools

import jax
import jax.numpy as jnp
from jax import lax
from jax.experimental import pallas as pl
from jax.experimental.pallas import tpu as pltpu
from jax.experimental.pallas import tpu_sc as plsc

N = 10000
E = 320000
D = 128
NC = 2
NS = 16
NW = NC * NS
B = 80
CPW = 125
PH = 64
DB = 80
DCPW = E // NW // DB
NPAD = NS * 640
RPS = 640
ZR = 128

_MESH = plsc.VectorSubcoreMesh(core_axis_name="c", subcore_axis_name="s")


@functools.partial(
    pl.kernel,
    out_type=jax.ShapeDtypeStruct((NC * NPAD,), jnp.float32),
    mesh=_MESH,
    scratch_types=[
        pltpu.VMEM((DCPW, DB), jnp.int32),
        pltpu.VMEM((DB,), jnp.float32),
        pltpu.VMEM((DB,), jnp.float32),
        pltpu.VMEM_SHARED((NPAD,), jnp.float32),
    ],
)
def _deg_call(dst_hbm, out_hbm, dstv, onesv, zerosv, acc):
    c = lax.axis_index("c")
    s = lax.axis_index("s")
    w = s * NC + c
    for i in range(DB // 16):
        onesv[pl.ds(i * 16, 16)] = jnp.ones((16,), jnp.float32)
        zerosv[pl.ds(i * 16, 16)] = jnp.zeros((16,), jnp.float32)
    for k in range(640 // DB):
        pltpu.sync_copy(zerosv, acc.at[pl.ds(s * 640 + k * DB, DB)])
    plsc.subcore_barrier()
    pltpu.sync_copy(dst_hbm.at[w], dstv)

    def body(j, carry):
        pltpu.sync_copy(onesv, acc.at[dstv.at[j]], add=True)
        return carry

    lax.fori_loop(0, DCPW, body, 0)
    plsc.subcore_barrier()
    pltpu.sync_copy(
        acc.at[pl.ds(s * 640, 640)], out_hbm.at[pl.ds(c * NPAD + s * 640, 640)]
    )


@functools.partial(
    pl.kernel,
    out_type=jax.ShapeDtypeStruct((NC, NPAD, D), jnp.float32),
    mesh=_MESH,
    scratch_types=[
        pltpu.VMEM((PH, B), jnp.int32),
        pltpu.VMEM((PH, B), jnp.int32),
        pltpu.VMEM((B, D), jnp.float32),
        pltpu.VMEM((B, D), jnp.float32),
        pltpu.VMEM_SHARED((NPAD, D), jnp.float32),
        pltpu.SemaphoreType.DMA,
        pltpu.SemaphoreType.DMA,
    ],
)
def _agg_call(g_hbm, src_hbm, dst_hbm, out_hbm, srcv, dstv, rows_a, rows_b, acc, r0, r1):
    c = lax.axis_index("c")
    s = lax.axis_index("s")
    w = s * NC + c
    zv = jnp.zeros((16,), jnp.float32)

    def zbody(i, carry):
        for j in range(D // 16):
            rows_a[i, pl.ds(j * 16, 16)] = zv
        return carry

    lax.fori_loop(0, B, zbody, 0)
    for k in range(RPS // B):
        pltpu.sync_copy(rows_a, acc.at[pl.ds(s * RPS + k * B, B)])
    plsc.subcore_barrier()

    def pair(ja, last):
        pltpu.async_copy(g_hbm.at[srcv.at[ja + 1]], rows_b, r1)
        pltpu.make_async_copy(g_hbm.at[srcv.at[ja]], rows_a, r0).wait()
        pltpu.sync_copy(rows_a, acc.at[dstv.at[ja]], add=True)
        if not last:
            pltpu.async_copy(g_hbm.at[srcv.at[ja + 2]], rows_a, r0)
        pltpu.make_async_copy(g_hbm.at[srcv.at[ja + 1]], rows_b, r1).wait()
        pltpu.sync_copy(rows_b, acc.at[dstv.at[ja + 1]], add=True)

    def single(j):
        pltpu.make_async_copy(g_hbm.at[srcv.at[j]], rows_a, r0).wait()
        pltpu.sync_copy(rows_a, acc.at[dstv.at[j]], add=True)

    pltpu.sync_copy(src_hbm.at[w, pl.ds(0, PH)], srcv)
    pltpu.sync_copy(dst_hbm.at[w, pl.ds(0, PH)], dstv)
    pltpu.async_copy(g_hbm.at[srcv.at[0]], rows_a, r0)

    def body_a(i, carry):
        pair(2 * i, False)
        return carry

    lax.fori_loop(0, PH // 2 - 1, body_a, 0)
    pair(PH - 2, True)
    nb = CPW - PH
    pltpu.sync_copy(src_hbm.at[w, pl.ds(PH, nb)], srcv.at[pl.ds(0, nb)])
    pltpu.sync_copy(dst_hbm.at[w, pl.ds(PH, nb)], dstv.at[pl.ds(0, nb)])
    pltpu.async_copy(g_hbm.at[srcv.at[0]], rows_a, r0)

    def body_b(i, carry):
        pair(2 * i, False)
        return carry

    lax.fori_loop(0, (nb - 3) // 2, body_b, 0)
    pair(nb - 3, True)
    pltpu.async_copy(g_hbm.at[srcv.at[nb - 1]], rows_a, r0)
    single(nb - 1)
    plsc.subcore_barrier()
    for k in range(RPS // ZR):
        base = s * RPS + k * ZR
        pltpu.sync_copy(acc.at[pl.ds(base, ZR)], out_hbm.at[c, pl.ds(base, ZR)])


BM = 1000


def _mm_body(x_ref, w_ref, o_ref):
    o_ref[...] = jnp.dot(x_ref[...], w_ref[...], preferred_element_type=jnp.float32)


def _mm(x, w):
    return pl.pallas_call(
        _mm_body,
        grid=(N // BM,),
        in_specs=[
            pl.BlockSpec((BM, D), lambda i: (i, 0)),
            pl.BlockSpec((D, D), lambda i: (0, 0)),
        ],
        out_specs=pl.BlockSpec((BM, D), lambda i: (i, 0)),
        out_shape=jax.ShapeDtypeStruct((N, D), jnp.float32),
    )(x, w)


def _prep_body(dega_ref, degb_ref, xw_ref, dinv_ref, g_ref):
    deg = dega_ref[...] + degb_ref[...]
    dinv = jnp.where(deg > 0, 1.0 / jnp.sqrt(jnp.maximum(deg, 1.0)), 0.0)
    dinv_ref[...] = dinv
    g_ref[...] = xw_ref[...] * dinv


def _prep(dega, degb, xw):
    return pl.pallas_call(
        _prep_body,
        grid=(N // BM,),
        in_specs=[
            pl.BlockSpec((BM, 1), lambda i: (i, 0)),
            pl.BlockSpec((BM, 1), lambda i: (i, 0)),
            pl.BlockSpec((BM, D), lambda i: (i, 0)),
        ],
        out_specs=[
            pl.BlockSpec((BM, 1), lambda i: (i, 0)),
            pl.BlockSpec((BM, D), lambda i: (i, 0)),
        ],
        out_shape=[
            jax.ShapeDtypeStruct((N, 1), jnp.float32),
            jax.ShapeDtypeStruct((N, D), jnp.float32),
        ],
    )(dega, degb, xw)


def _mid_body(s1_ref, dinv_ref, b1_ref, w2_ref, h1_ref, g2_ref):
    stot = s1_ref[0] + s1_ref[1]
    h1 = jnp.maximum(stot * dinv_ref[...] + b1_ref[...], 0.0)
    h1_ref[...] = h1
    g2_ref[...] = (
        jnp.dot(h1, w2_ref[...], preferred_element_type=jnp.float32) * dinv_ref[...]
    )


def _mid(s1, dinv, b1, w2):
    return pl.pallas_call(
        _mid_body,
        grid=(N // BM,),
        in_specs=[
            pl.BlockSpec((NC, BM, D), lambda i: (0, i, 0)),
            pl.BlockSpec((BM, 1), lambda i: (i, 0)),
            pl.BlockSpec((1, D), lambda i: (0, 0)),
            pl.BlockSpec((D, D), lambda i: (0, 0)),
        ],
        out_specs=[
            pl.BlockSpec((BM, D), lambda i: (i, 0)),
            pl.BlockSpec((BM, D), lambda i: (i, 0)),
        ],
        out_shape=[
            jax.ShapeDtypeStruct((N, D), jnp.float32),
            jax.ShapeDtypeStruct((N, D), jnp.float32),
        ],
    )(s1, dinv, b1, w2)


def _out_body(s2_ref, dinv_ref, b2_ref, h2_ref):
    h2_ref[...] = (s2_ref[0] + s2_ref[1]) * dinv_ref[...] + b2_ref[...]


def _out(s2, dinv, b2):
    return pl.pallas_call(
        _out_body,
        grid=(N // BM,),
        in_specs=[
            pl.BlockSpec((NC, BM, D), lambda i: (0, i, 0)),
            pl.BlockSpec((BM, 1), lambda i: (i, 0)),
            pl.BlockSpec((1, D), lambda i: (0, 0)),
        ],
        out_specs=pl.BlockSpec((BM, D), lambda i: (i, 0)),
        out_shape=jax.ShapeDtypeStruct((N, D), jnp.float32),
    )(s2, dinv, b2)


def kernel(x, edge_index, W1, b1, W2, b2):
    src3 = edge_index[0].reshape(NW, CPW, B)
    dst3 = edge_index[1].reshape(NW, CPW, B)
    deg1d = _deg_call(dst3)
    dega = deg1d[:N].reshape(N, 1)
    degb = deg1d[NPAD : NPAD + N].reshape(N, 1)
    xw1 = _mm(x, W1)
    dinv, g1 = _prep(dega, degb, xw1)
    s1 = _agg_call(g1, src3, dst3)
    h1, g2 = _mid(s1, dinv, b1.reshape(1, D), W2)
    s2 = _agg_call(g2, src3, dst3)
    h2 = _out(s2, dinv, b2.reshape(1, D))
    return (h1, h2)
